# Initial kernel scaffold; baseline (speedup 1.0000x reference)
import functools

import jax
import jax.numpy as jnp
from jax import lax
from jax.experimental import pallas as pl
from jax.experimental.pallas import tpu as pltpu
from jax.experimental.pallas import tpu_sc as plsc

_NC = 2
_NS = 16
_NW = _NC * _NS
_CH = 128


def _make_seg_sum(n_nodes, width, chunks_per_tile):
  acc_rows = n_nodes + 16
  zchunk = acc_rows // _NS
  ochunk = n_nodes // _NS
  mesh = plsc.VectorSubcoreMesh(core_axis_name="c", subcore_axis_name="s")

  @functools.partial(
      pl.kernel,
      out_type=jax.ShapeDtypeStruct((_NC, n_nodes, width), jnp.float32),
      mesh=mesh,
      scratch_types=[
          pltpu.VMEM_SHARED((acc_rows, width), jnp.float32),
          pltpu.VMEM((chunks_per_tile, _CH), jnp.int32),
          pltpu.VMEM((chunks_per_tile, _CH), jnp.int32),
          pltpu.VMEM((_CH, width), jnp.float32),
          pltpu.SemaphoreType.DMA,
      ],
  )
  def seg_kernel(table, srcb, dstb, zrows, out, acc, src_v, dst_v, rows_v, sem):
    c = lax.axis_index("c")
    s = lax.axis_index("s")
    wid = c * _NS + s
    pltpu.sync_copy(zrows, acc.at[pl.ds(s * zchunk, zchunk)])
    pltpu.sync_copy(srcb.at[wid], src_v)
    pltpu.sync_copy(dstb.at[wid], dst_v)
    plsc.subcore_barrier()

    def body(j, carry):
      pltpu.async_copy(table.at[src_v.at[j]], rows_v, sem).wait()
      pltpu.sync_copy(rows_v, acc.at[dst_v.at[j]], add=True)
      return carry

    lax.fori_loop(0, chunks_per_tile, body, 0)
    plsc.subcore_barrier()
    pltpu.sync_copy(acc.at[pl.ds(s * ochunk, ochunk)],
                    out.at[c, pl.ds(s * ochunk, ochunk)])

  return seg_kernel


def _lin_body(x_ref, w_ref, b_ref, o_ref):
  o_ref[...] = lax.dot_general(
      x_ref[...], w_ref[...], (((1,), (1,)), ((), ())),
      preferred_element_type=jnp.float32) + b_ref[...]


def _mid_body(s0_ref, s1_ref, c0_ref, c1_ref, q_ref, w_ref, b_ref, o_ref):
  cnt = jnp.maximum(c0_ref[...] + c1_ref[...], 1.0)
  h = jnp.maximum((s0_ref[...] + s1_ref[...]) / cnt + q_ref[...], 0.0)
  o_ref[...] = lax.dot_general(
      h, w_ref[...], (((1,), (1,)), ((), ())),
      preferred_element_type=jnp.float32) + b_ref[...]


def _out_body(s0_ref, s1_ref, c0_ref, c1_ref, q_ref, o_ref):
  cnt = jnp.maximum(c0_ref[...] + c1_ref[...], 1.0)
  z = (s0_ref[...] + s1_ref[...]) / cnt + q_ref[...]
  z = z - jnp.max(z, axis=1, keepdims=True)
  o_ref[...] = z - jnp.log(jnp.sum(jnp.exp(z), axis=1, keepdims=True))


def kernel(x, edge_index, W1l, b1l, W1r, W2l, b2l, W2r):
  n = x.shape[0]
  e = edge_index.shape[1]
  hid = W1l.shape[0]
  out_ch = W2l.shape[0]

  src = edge_index[0].astype(jnp.int32)
  dst = edge_index[1].astype(jnp.int32)
  per = _NW * _CH
  chunks = (e + per - 1) // per
  pad = chunks * per - e
  srcb = jnp.concatenate([src, jnp.zeros((pad,), jnp.int32)]).reshape(
      _NW, chunks, _CH)
  dstb = jnp.concatenate([dst, jnp.full((pad,), n, jnp.int32)]).reshape(
      _NW, chunks, _CH)

  w1 = jnp.concatenate([W1l, W1r], axis=0)
  bias1 = jnp.concatenate([jnp.zeros((hid,), jnp.float32), b1l])[None, :]
  pq1 = pl.pallas_call(
      _lin_body,
      out_shape=jax.ShapeDtypeStruct((n, 2 * hid), jnp.float32),
  )(x, w1, bias1)

  w_tab = 2 * hid
  table1 = jnp.concatenate(
      [pq1[:, :hid], jnp.ones((n, 1), jnp.float32),
       jnp.zeros((n, w_tab - hid - 1), jnp.float32)], axis=1)

  zrows32 = jnp.zeros(((n + 16) // _NS, w_tab), jnp.float32)
  seg32 = _make_seg_sum(n, w_tab, chunks)
  part1 = seg32(table1, srcb, dstb, zrows32)

  s0 = part1[0, :, :hid]
  s1 = part1[1, :, :hid]
  c0 = part1[0, :, hid:hid + 1]
  c1 = part1[1, :, hid:hid + 1]

  w2 = jnp.concatenate([W2l, W2r], axis=0)
  bias2 = jnp.concatenate([jnp.zeros((out_ch,), jnp.float32), b2l])[None, :]
  pq2 = pl.pallas_call(
      _mid_body,
      out_shape=jax.ShapeDtypeStruct((n, 2 * out_ch), jnp.float32),
  )(s0, s1, c0, c1, pq1[:, hid:], w2, bias2)

  table2 = pq2[:, :out_ch]
  zrows16 = jnp.zeros(((n + 16) // _NS, out_ch), jnp.float32)
  seg16 = _make_seg_sum(n, out_ch, chunks)
  part2 = seg16(table2, srcb, dstb, zrows16)

  out = pl.pallas_call(
      _out_body,
      out_shape=jax.ShapeDtypeStruct((n, out_ch), jnp.float32),
  )(part2[0], part2[1], c0, c1, pq2[:, out_ch:])
  return out


# speedup vs baseline: 10.6693x; 10.6693x over previous
import functools

import jax
import jax.numpy as jnp
from jax import lax
from jax.experimental import pallas as pl
from jax.experimental.pallas import tpu as pltpu
from jax.experimental.pallas import tpu_sc as plsc

_NC = 2
_NS = 16
_NW = _NC * _NS
_CH = 128


def _make_seg_sum(n_nodes, width, chunks_per_tile):
  acc_rows = (n_nodes // 128 + 1) * 128
  zchunk = acc_rows // _NS
  mesh = plsc.VectorSubcoreMesh(core_axis_name="c", subcore_axis_name="s")

  @functools.partial(
      pl.kernel,
      out_type=jax.ShapeDtypeStruct((_NC, acc_rows, width), jnp.float32),
      mesh=mesh,
      scratch_types=[
          pltpu.VMEM_SHARED((acc_rows, width), jnp.float32),
          pltpu.VMEM((chunks_per_tile, _CH), jnp.int32),
          pltpu.VMEM((chunks_per_tile, _CH), jnp.int32),
          pltpu.VMEM((_CH, width), jnp.float32),
          pltpu.SemaphoreType.DMA,
      ],
      compiler_params=pltpu.CompilerParams(use_tc_tiling_on_sc=False),
  )
  def seg_kernel(table, srcb, dstb, zrows, out, acc, src_v, dst_v, rows_v, sem):
    c = lax.axis_index("c")
    s = lax.axis_index("s")
    wid = c * _NS + s
    pltpu.sync_copy(zrows, acc.at[pl.ds(s * zchunk, zchunk)])
    pltpu.sync_copy(srcb.at[wid], src_v)
    pltpu.sync_copy(dstb.at[wid], dst_v)
    plsc.subcore_barrier()

    def body(j, carry):
      pltpu.async_copy(table.at[src_v.at[j]], rows_v, sem).wait()
      pltpu.sync_copy(rows_v, acc.at[dst_v.at[j]], add=True)
      return carry

    lax.fori_loop(0, chunks_per_tile, body, 0)
    plsc.subcore_barrier()
    pltpu.sync_copy(acc.at[pl.ds(s * zchunk, zchunk)],
                    out.at[c, pl.ds(s * zchunk, zchunk)])

  return seg_kernel


def _lin_body(x_ref, w_ref, b_ref, o_ref):
  o_ref[...] = lax.dot_general(
      x_ref[...], w_ref[...], (((1,), (1,)), ((), ())),
      preferred_element_type=jnp.float32) + b_ref[...]


def _mid_body(s0_ref, s1_ref, c0_ref, c1_ref, q_ref, w_ref, b_ref, o_ref):
  cnt = jnp.maximum(c0_ref[...] + c1_ref[...], 1.0)
  h = jnp.maximum((s0_ref[...] + s1_ref[...]) / cnt + q_ref[...], 0.0)
  o_ref[...] = lax.dot_general(
      h, w_ref[...], (((1,), (1,)), ((), ())),
      preferred_element_type=jnp.float32) + b_ref[...]


def _out_body(s0_ref, s1_ref, c0_ref, c1_ref, q_ref, o_ref):
  cnt = jnp.maximum(c0_ref[...] + c1_ref[...], 1.0)
  z = (s0_ref[...] + s1_ref[...]) / cnt + q_ref[...]
  z = z - jnp.max(z, axis=1, keepdims=True)
  o_ref[...] = z - jnp.log(jnp.sum(jnp.exp(z), axis=1, keepdims=True))


def kernel(x, edge_index, W1l, b1l, W1r, W2l, b2l, W2r):
  n = x.shape[0]
  e = edge_index.shape[1]
  hid = W1l.shape[0]
  out_ch = W2l.shape[0]

  src = edge_index[0].astype(jnp.int32)
  dst = edge_index[1].astype(jnp.int32)
  per = _NW * _CH
  chunks = (e + per - 1) // per
  pad = chunks * per - e
  srcb = jnp.concatenate([src, jnp.zeros((pad,), jnp.int32)]).reshape(
      _NW, chunks, _CH)
  dstb = jnp.concatenate([dst, jnp.full((pad,), n, jnp.int32)]).reshape(
      _NW, chunks, _CH)

  w1 = jnp.concatenate([W1l, W1r], axis=0)
  bias1 = jnp.concatenate([jnp.zeros((hid,), jnp.float32), b1l])[None, :]
  pq1 = pl.pallas_call(
      _lin_body,
      out_shape=jax.ShapeDtypeStruct((n, 2 * hid), jnp.float32),
  )(x, w1, bias1)

  w_tab = 2 * hid
  table1 = jnp.concatenate(
      [pq1[:, :hid], jnp.ones((n, 1), jnp.float32),
       jnp.zeros((n, w_tab - hid - 1), jnp.float32)], axis=1)

  npad = (n // 128 + 1) * 128
  zrows32 = jnp.zeros((npad // _NS, w_tab), jnp.float32)
  seg32 = _make_seg_sum(n, w_tab, chunks)
  part1 = seg32(table1, srcb, dstb, zrows32)

  s0 = part1[0, :n, :hid]
  s1 = part1[1, :n, :hid]
  c0 = part1[0, :n, hid:hid + 1]
  c1 = part1[1, :n, hid:hid + 1]

  w2 = jnp.concatenate([W2l, W2r], axis=0)
  bias2 = jnp.concatenate([jnp.zeros((out_ch,), jnp.float32), b2l])[None, :]
  pq2 = pl.pallas_call(
      _mid_body,
      out_shape=jax.ShapeDtypeStruct((n, 2 * out_ch), jnp.float32),
  )(s0, s1, c0, c1, pq1[:, hid:], w2, bias2)

  table2 = pq2[:, :out_ch]
  zrows16 = jnp.zeros((npad // _NS, out_ch), jnp.float32)
  seg16 = _make_seg_sum(n, out_ch, chunks)
  part2 = seg16(table2, srcb, dstb, zrows16)

  out = pl.pallas_call(
      _out_body,
      out_shape=jax.ShapeDtypeStruct((n, out_ch), jnp.float32),
  )(part2[0, :n], part2[1, :n], c0, c1, pq2[:, out_ch:])
  return out
